# Initial kernel scaffold; baseline (speedup 1.0000x reference)
import jax
import jax.numpy as jnp
from jax.experimental import pallas as pl


def kernel(data, label, center):
    raise NotImplementedError("write your pallas kernel here")


# speedup vs baseline: 3.1452x; 3.1452x over previous
import functools

import jax
import jax.numpy as jnp
from jax import lax
from jax.experimental import pallas as pl
from jax.experimental.pallas import tpu as pltpu
from jax.experimental.pallas import tpu_sc as plsc

CLS = 1000
CBINS = 1024
FEAT = 64
N = 1_000_000

NC, NS, L = 2, 16, 16
NW = NC * NS

Q = 31_264
QL = N - (NW - 1) * Q

BLK = 320
NBLK = N // BLK
GPB = BLK // L

_mesh = plsc.VectorSubcoreMesh(core_axis_name="c", subcore_axis_name="s")
_params = pltpu.CompilerParams(needs_layout_passes=False)


@functools.partial(
    pl.kernel,
    out_type=jax.ShapeDtypeStruct((NW, CBINS), jnp.float32),
    mesh=_mesh,
    scratch_types=[
        pltpu.VMEM((Q,), jnp.float32),
        pltpu.VMEM((L * CBINS,), jnp.float32),
        pltpu.VMEM((CBINS,), jnp.float32),
    ],
    compiler_params=_params,
)
def _hist_kernel(lbl_hbm, out_hbm, lbl_v, h_v, cnt_v):
    c = lax.axis_index("c")
    s = lax.axis_index("s")
    wid = s * NC + c

    zeros = jnp.zeros((L,), jnp.float32)

    def zero_body(j, _):
        h_v[pl.ds(j * L, L)] = zeros
        return 0

    lax.fori_loop(0, (L * CBINS) // L, zero_body, 0)

    @pl.when(wid < NW - 1)
    def _():
        pltpu.sync_copy(lbl_hbm.at[pl.ds(wid * Q, Q)], lbl_v)

    @pl.when(wid == NW - 1)
    def _():
        pltpu.sync_copy(lbl_hbm.at[pl.ds((NW - 1) * Q, QL)],
                        lbl_v.at[pl.ds(0, QL)])

    rows = lax.iota(jnp.int32, L) * CBINS
    ones = jnp.ones((L,), jnp.float32)
    ng = jnp.where(wid < NW - 1, Q // L, QL // L)

    def scat_body(g, _):
        lv = lbl_v[pl.ds(g * L, L)].astype(jnp.int32)
        plsc.addupdate_scatter(h_v, [rows + lv], ones)
        return 0

    lax.fori_loop(0, ng, scat_body, 0)

    def red_body(j, _):
        acc = h_v[pl.ds(j * L, L)]
        for r in range(1, L):
            acc = acc + h_v[pl.ds(r * CBINS + j * L, L)]
        cnt_v[pl.ds(j * L, L)] = acc
        return 0

    lax.fori_loop(0, CBINS // L, red_body, 0)
    pltpu.sync_copy(cnt_v, out_hbm.at[wid])


@functools.partial(
    pl.kernel,
    out_type=jax.ShapeDtypeStruct((NW, L), jnp.float32),
    mesh=_mesh,
    scratch_types=[
        pltpu.VMEM((CLS * FEAT,), jnp.float32),
        pltpu.VMEM((NW * CBINS,), jnp.float32),
        pltpu.VMEM((CBINS,), jnp.float32),
        pltpu.VMEM((BLK * FEAT,), jnp.float32),
        pltpu.VMEM((BLK,), jnp.float32),
        pltpu.VMEM((L,), jnp.float32),
    ],
    compiler_params=_params,
)
def _loss_kernel(data_hbm, lbl_hbm, cen_hbm, cnt_hbm, out_hbm,
                 cen_v, c32_v, cnt_v, dat_v, lb_v, res_v):
    c = lax.axis_index("c")
    s = lax.axis_index("s")
    wid = s * NC + c

    pltpu.sync_copy(cen_hbm, cen_v)
    pltpu.sync_copy(cnt_hbm, c32_v)

    def comb_body(j, _):
        acc = c32_v[pl.ds(j * L, L)]
        for w in range(1, NW):
            acc = acc + c32_v[pl.ds(w * CBINS + j * L, L)]
        cnt_v[pl.ds(j * L, L)] = acc
        return 0

    lax.fori_loop(0, CBINS // L, comb_body, 0)

    riota = lax.iota(jnp.int32, L) * FEAT
    half = jnp.float32(0.5)
    three_half = jnp.float32(1.5)

    def blk_body(k, lsum):
        b = wid + k * NW
        row0 = b * BLK
        pltpu.sync_copy(data_hbm.at[pl.ds(row0 * FEAT, BLK * FEAT)], dat_v)
        pltpu.sync_copy(lbl_hbm.at[pl.ds(row0, BLK)], lb_v)

        def grp_body(g, ls):
            lv = lb_v[pl.ds(g * L, L)].astype(jnp.int32)
            cw = plsc.load_gather(cnt_v, [lv])
            cidx = lv * FEAT
            didx = riota + g * (L * FEAT)
            acc = jnp.zeros((L,), jnp.float32)
            for f in range(FEAT):
                dv = plsc.load_gather(dat_v, [didx + f])
                cv = plsc.load_gather(cen_v, [cidx + f])
                t = dv - cv
                acc = acc + t * t
            x = jnp.maximum(acc, jnp.float32(1e-30))
            i = plsc.bitcast(x, jnp.int32)
            i = jnp.int32(0x5F3759DF) - lax.shift_right_logical(i, 1)
            y = plsc.bitcast(i, jnp.float32)
            for _ in range(3):
                y = y * (three_half - half * x * y * y)
            dist = jnp.where(acc > 0.0, x * y, jnp.float32(0.0))
            return ls + dist / cw

        return lax.fori_loop(0, GPB, grp_body, lsum)

    nmine = (NBLK - wid + NW - 1) // NW
    lsum = lax.fori_loop(0, nmine, blk_body, jnp.zeros((L,), jnp.float32))
    res_v[...] = lsum
    pltpu.sync_copy(res_v, out_hbm.at[wid])


def kernel(data, label, center):
    counts = _hist_kernel(label)
    parts = _loss_kernel(data.reshape(-1), label, center.reshape(-1),
                         counts.reshape(-1))
    return jnp.sum(parts)


# speedup vs baseline: 7.4485x; 2.3682x over previous
import functools

import jax
import jax.numpy as jnp
from jax import lax
from jax.experimental import pallas as pl
from jax.experimental.pallas import tpu as pltpu
from jax.experimental.pallas import tpu_sc as plsc

CLS = 1000
CBINS = 1024
FEAT = 64
N = 1_000_000

NC, NS, L = 2, 16, 16
NW = NC * NS

Q = 31_264
QL = N - (NW - 1) * Q

BLK = 320
NBLK = N // BLK
GPB = BLK // L

_mesh = plsc.VectorSubcoreMesh(core_axis_name="c", subcore_axis_name="s")
_params = pltpu.CompilerParams(needs_layout_passes=False)


@functools.partial(
    pl.kernel,
    out_type=jax.ShapeDtypeStruct((NW, CBINS), jnp.float32),
    mesh=_mesh,
    scratch_types=[
        pltpu.VMEM((Q,), jnp.float32),
        pltpu.VMEM((L * CBINS,), jnp.float32),
        pltpu.VMEM((CBINS,), jnp.float32),
    ],
    compiler_params=_params,
)
def _hist_kernel(lbl_hbm, out_hbm, lbl_v, h_v, cnt_v):
    c = lax.axis_index("c")
    s = lax.axis_index("s")
    wid = s * NC + c

    zeros = jnp.zeros((L,), jnp.float32)

    def zero_body(j, _):
        h_v[pl.ds(j * L, L)] = zeros
        return 0

    lax.fori_loop(0, (L * CBINS) // L, zero_body, 0)

    @pl.when(wid < NW - 1)
    def _():
        pltpu.sync_copy(lbl_hbm.at[pl.ds(wid * Q, Q)], lbl_v)

    @pl.when(wid == NW - 1)
    def _():
        pltpu.sync_copy(lbl_hbm.at[pl.ds((NW - 1) * Q, QL)],
                        lbl_v.at[pl.ds(0, QL)])

    rows = lax.iota(jnp.int32, L) * CBINS
    ones = jnp.ones((L,), jnp.float32)
    ng = jnp.where(wid < NW - 1, Q // L, QL // L)

    def scat_body(g, _):
        lv = lbl_v[pl.ds(g * L, L)].astype(jnp.int32)
        plsc.addupdate_scatter(h_v, [rows + lv], ones)
        return 0

    lax.fori_loop(0, ng, scat_body, 0)

    def red_body(j, _):
        acc = h_v[pl.ds(j * L, L)]
        for r in range(1, L):
            acc = acc + h_v[pl.ds(r * CBINS + j * L, L)]
        cnt_v[pl.ds(j * L, L)] = acc
        return 0

    lax.fori_loop(0, CBINS // L, red_body, 0)
    pltpu.sync_copy(cnt_v, out_hbm.at[wid])


@functools.partial(
    pl.kernel,
    out_type=jax.ShapeDtypeStruct((NW, L), jnp.float32),
    mesh=_mesh,
    scratch_types=[
        pltpu.VMEM((CLS * FEAT,), jnp.float32),
        pltpu.VMEM((NW * CBINS,), jnp.float32),
        pltpu.VMEM((CBINS,), jnp.float32),
        pltpu.VMEM((BLK * FEAT,), jnp.float32),
        pltpu.VMEM((BLK,), jnp.float32),
        pltpu.VMEM((L,), jnp.float32),
    ],
    compiler_params=_params,
)
def _loss_kernel(data_hbm, lbl_hbm, cen_hbm, cnt_hbm, out_hbm,
                 cen_v, c32_v, cnt_v, dat_v, lb_v, res_v):
    c = lax.axis_index("c")
    s = lax.axis_index("s")
    wid = s * NC + c

    pltpu.sync_copy(cen_hbm, cen_v)
    pltpu.sync_copy(cnt_hbm, c32_v)

    def comb_body(j, _):
        acc = c32_v[pl.ds(j * L, L)]
        for w in range(1, NW):
            acc = acc + c32_v[pl.ds(w * CBINS + j * L, L)]
        cnt_v[pl.ds(j * L, L)] = acc
        return 0

    lax.fori_loop(0, CBINS // L, comb_body, 0)

    siota = lax.iota(jnp.int32, L)
    riota = siota * FEAT
    half = jnp.float32(0.5)
    three_half = jnp.float32(1.5)

    def blk_body(k, lsum):
        b = wid + k * NW
        row0 = b * BLK
        pltpu.sync_copy(data_hbm.at[pl.ds(row0 * FEAT, BLK * FEAT)], dat_v)
        pltpu.sync_copy(lbl_hbm.at[pl.ds(row0, BLK)], lb_v)

        def grp_body(g, ls):
            lv = lb_v[pl.ds(g * L, L)].astype(jnp.int32)
            cw = plsc.load_gather(cnt_v, [lv])
            cidx = lv * FEAT
            didx = riota + g * (L * FEAT)
            acc = jnp.zeros((L,), jnp.float32)
            for f in range(FEAT):
                fv = (siota + f) & (FEAT - 1)
                dv = plsc.load_gather(dat_v, [didx + fv])
                cv = plsc.load_gather(cen_v, [cidx + fv])
                t = dv - cv
                acc = acc + t * t
            x = jnp.maximum(acc, jnp.float32(1e-30))
            i = plsc.bitcast(x, jnp.int32)
            i = jnp.int32(0x5F3759DF) - lax.shift_right_logical(i, 1)
            y = plsc.bitcast(i, jnp.float32)
            for _ in range(3):
                y = y * (three_half - half * x * y * y)
            dist = jnp.where(acc > 0.0, x * y, jnp.float32(0.0))
            return ls + dist / cw

        return lax.fori_loop(0, GPB, grp_body, lsum)

    nmine = (NBLK - wid + NW - 1) // NW
    lsum = lax.fori_loop(0, nmine, blk_body, jnp.zeros((L,), jnp.float32))
    res_v[...] = lsum
    pltpu.sync_copy(res_v, out_hbm.at[wid])


def kernel(data, label, center):
    counts = _hist_kernel(label)
    parts = _loss_kernel(data.reshape(-1), label, center.reshape(-1),
                         counts.reshape(-1))
    return jnp.sum(parts)
